# baseline (speedup 1.0000x reference)
import functools

import jax
import jax.numpy as jnp
from jax import lax
from jax.experimental import pallas as pl
from jax.experimental.pallas import tpu as pltpu
from jax.experimental.pallas import tpu_sc as plsc

_BATCH = 16384
_HIST = 50
_EMBED = 32
_NC = 2
_NS = 16
_NW = _NC * _NS
_RPW = _BATCH // _NW
_XB = 128
_NBLK = _RPW // _XB
_NB = 4
_K = 2


def _gather_body(x_hbm, w_hbm, out_hbm, idx_v, rows_v, nrw_v, *sems):
    gsem = sems[:_NB]
    ssem = sems[_NB:]
    wid = lax.axis_index("s") * _NC + lax.axis_index("c")
    row0 = wid * _RPW

    for blk in range(_NBLK):
        base = row0 + blk * _XB

        def gather(g, b):
            pltpu.async_copy(w_hbm.at[idx_v.at[g]], rows_v.at[b], gsem[b])

        def wait_gather(b):
            pltpu.make_async_copy(w_hbm.at[idx_v.at[0]], rows_v.at[b],
                                  gsem[b]).wait()

        def store(g, b):
            pltpu.async_copy(nrw_v.at[b], out_hbm.at[base + g], ssem[b])

        def wait_store(b):
            pltpu.make_async_copy(nrw_v.at[b], out_hbm.at[0],
                                  ssem[b]).wait()

        def repack(b):
            def rp(h, c):
                nrw_v[b, h, pl.ds(0, 16)] = rows_v[b, h, pl.ds(0, 16)]
                nrw_v[b, h, pl.ds(16, 16)] = rows_v[b, h, pl.ds(16, 16)]
                return c
            lax.fori_loop(0, _HIST, rp, 0)

        def body(g, b, bk, with_ssem_wait):
            if with_ssem_wait:
                wait_store(bk)
            gather(g + _K, bk)
            wait_gather(b)
            repack(b)
            store(g, b)

        pltpu.sync_copy(x_hbm.at[pl.ds(base, _XB)], idx_v)
        for g in range(_K):
            gather(g, g)
        for g in range(_NB - _K):
            body(g, g % _NB, (g + _K) % _NB, False)
        for g in range(_NB - _K, _NB):
            body(g, g % _NB, (g + _K) % _NB, True)

        def outer(i, carry):
            for b in range(_NB):
                g = i * _NB + b
                body(g, b, (b + _K) % _NB, True)
            return carry

        lax.fori_loop(1, (_XB - _K) // _NB, outer, 0)

        for g in range(_XB - _NB, _XB - _K):
            body(g, g % _NB, (g + _K) % _NB, True)
        for g in range(_XB - _K, _XB):
            wait_gather(g % _NB)
            repack(g % _NB)
            store(g, g % _NB)
        for g in range(_XB - _NB, _XB):
            wait_store(g % _NB)


@jax.jit
def _embed(x, w_wide):
    k = functools.partial(
        pl.kernel,
        mesh=plsc.VectorSubcoreMesh(core_axis_name="c", subcore_axis_name="s"),
        out_type=jax.ShapeDtypeStruct((_BATCH, _HIST, _EMBED), jnp.float32),
        scratch_types=[
            pltpu.VMEM((_XB, _HIST), jnp.int32),
            pltpu.VMEM((_NB, _HIST, 128), jnp.float32),
            pltpu.VMEM((_NB, _HIST, _EMBED), jnp.float32),
        ] + [pltpu.SemaphoreType.DMA] * (2 * _NB),
    )(_gather_body)
    return k(x, w_wide)


def kernel(x, W):
    w_wide = jnp.pad(W, ((0, 0), (0, 128 - _EMBED)))
    return _embed(x, w_wide)


# speedup vs baseline: 1.0009x; 1.0009x over previous
import functools

import jax
import jax.numpy as jnp
from jax import lax
from jax.experimental import pallas as pl
from jax.experimental.pallas import tpu as pltpu
from jax.experimental.pallas import tpu_sc as plsc

_BATCH = 16384
_HIST = 50
_EMBED = 32
_NC = 2
_NS = 16
_NW = _NC * _NS
_RPW = _BATCH // _NW
_XB = 128
_NBLK = _RPW // _XB
_NB = 4
_K = 2


def _gather_body(x_hbm, w_hbm, out_hbm, idx_v, rows_v, nrw_v, *sems):
    gsem = sems[:_NB]
    ssem = sems[_NB:]
    wid = lax.axis_index("s") * _NC + lax.axis_index("c")
    row0 = wid * _RPW

    for blk in range(_NBLK):
        base = row0 + blk * _XB

        def gather(g, b):
            pltpu.async_copy(w_hbm.at[idx_v.at[g]], rows_v.at[b], gsem[b])

        def wait_gather(b):
            pltpu.make_async_copy(w_hbm.at[idx_v.at[0]], rows_v.at[b],
                                  gsem[b]).wait()

        def store(g, b):
            pltpu.async_copy(nrw_v.at[b], out_hbm.at[base + g], ssem[b])

        def wait_store(b):
            pltpu.make_async_copy(nrw_v.at[b], out_hbm.at[0],
                                  ssem[b]).wait()

        def repack(b):
            def rp(i, c):
                h0 = i * 10
                for j in range(10):
                    nrw_v[b, h0 + j, pl.ds(0, 16)] = (
                        rows_v[b, h0 + j, pl.ds(0, 16)])
                    nrw_v[b, h0 + j, pl.ds(16, 16)] = (
                        rows_v[b, h0 + j, pl.ds(16, 16)])
                return c
            lax.fori_loop(0, _HIST // 10, rp, 0)

        def body(g, b, bk, with_ssem_wait):
            if with_ssem_wait:
                wait_store(bk)
            gather(g + _K, bk)
            wait_gather(b)
            repack(b)
            store(g, b)

        pltpu.sync_copy(x_hbm.at[pl.ds(base, _XB)], idx_v)
        for g in range(_K):
            gather(g, g)
        for g in range(_NB - _K):
            body(g, g % _NB, (g + _K) % _NB, False)
        for g in range(_NB - _K, _NB):
            body(g, g % _NB, (g + _K) % _NB, True)

        def outer(i, carry):
            for b in range(_NB):
                g = i * _NB + b
                body(g, b, (b + _K) % _NB, True)
            return carry

        lax.fori_loop(1, (_XB - _K) // _NB, outer, 0)

        for g in range(_XB - _NB, _XB - _K):
            body(g, g % _NB, (g + _K) % _NB, True)
        for g in range(_XB - _K, _XB):
            wait_gather(g % _NB)
            repack(g % _NB)
            store(g, g % _NB)
        for g in range(_XB - _NB, _XB):
            wait_store(g % _NB)


@jax.jit
def _embed(x, w_wide):
    k = functools.partial(
        pl.kernel,
        mesh=plsc.VectorSubcoreMesh(core_axis_name="c", subcore_axis_name="s"),
        out_type=jax.ShapeDtypeStruct((_BATCH, _HIST, _EMBED), jnp.float32),
        scratch_types=[
            pltpu.VMEM((_XB, _HIST), jnp.int32),
            pltpu.VMEM((_NB, _HIST, 128), jnp.float32),
            pltpu.VMEM((_NB, _HIST, _EMBED), jnp.float32),
        ] + [pltpu.SemaphoreType.DMA] * (2 * _NB),
    )(_gather_body)
    return k(x, w_wide)


def kernel(x, W):
    w_wide = jnp.pad(W, ((0, 0), (0, 128 - _EMBED)))
    return _embed(x, w_wide)


# speedup vs baseline: 1.0074x; 1.0065x over previous
import functools

import jax
import jax.numpy as jnp
from jax import lax
from jax.experimental import pallas as pl
from jax.experimental.pallas import tpu as pltpu
from jax.experimental.pallas import tpu_sc as plsc

_BATCH = 16384
_HIST = 50
_EMBED = 32
_NC = 2
_NS = 16
_NW = _NC * _NS
_RPW = _BATCH // _NW
_XB = 128
_NBLK = _RPW // _XB
_NB = 8
_K = 6


def _gather_body(x_hbm, w_hbm, out_hbm, idx_v, rows_v, nrw_v, *sems):
    gsem = sems[:_NB]
    ssem = sems[_NB:]
    wid = lax.axis_index("s") * _NC + lax.axis_index("c")
    row0 = wid * _RPW

    for blk in range(_NBLK):
        base = row0 + blk * _XB

        def gather(g, b):
            pltpu.async_copy(w_hbm.at[idx_v.at[g]], rows_v.at[b], gsem[b])

        def wait_gather(b):
            pltpu.make_async_copy(w_hbm.at[idx_v.at[0]], rows_v.at[b],
                                  gsem[b]).wait()

        def store(g, b):
            pltpu.async_copy(nrw_v.at[b], out_hbm.at[base + g], ssem[b])

        def wait_store(b):
            pltpu.make_async_copy(nrw_v.at[b], out_hbm.at[0],
                                  ssem[b]).wait()

        def repack(b):
            def rp(i, c):
                h0 = i * 10
                for j in range(10):
                    nrw_v[b, h0 + j, pl.ds(0, 16)] = (
                        rows_v[b, h0 + j, pl.ds(0, 16)])
                    nrw_v[b, h0 + j, pl.ds(16, 16)] = (
                        rows_v[b, h0 + j, pl.ds(16, 16)])
                return c
            lax.fori_loop(0, _HIST // 10, rp, 0)

        def body(g, b, bk, with_ssem_wait):
            if with_ssem_wait:
                wait_store(bk)
            gather(g + _K, bk)
            wait_gather(b)
            repack(b)
            store(g, b)

        pltpu.sync_copy(x_hbm.at[pl.ds(base, _XB)], idx_v)
        for g in range(_K):
            gather(g, g)
        for g in range(_NB - _K):
            body(g, g % _NB, (g + _K) % _NB, False)
        for g in range(_NB - _K, _NB):
            body(g, g % _NB, (g + _K) % _NB, True)

        def outer(i, carry):
            for b in range(_NB):
                g = i * _NB + b
                body(g, b, (b + _K) % _NB, True)
            return carry

        lax.fori_loop(1, (_XB - _K) // _NB, outer, 0)

        for g in range(_XB - _NB, _XB - _K):
            body(g, g % _NB, (g + _K) % _NB, True)
        for g in range(_XB - _K, _XB):
            wait_gather(g % _NB)
            repack(g % _NB)
            store(g, g % _NB)
        for g in range(_XB - _NB, _XB):
            wait_store(g % _NB)


@jax.jit
def _embed(x, w_wide):
    k = functools.partial(
        pl.kernel,
        mesh=plsc.VectorSubcoreMesh(core_axis_name="c", subcore_axis_name="s"),
        out_type=jax.ShapeDtypeStruct((_BATCH, _HIST, _EMBED), jnp.float32),
        scratch_types=[
            pltpu.VMEM((_XB, _HIST), jnp.int32),
            pltpu.VMEM((_NB, _HIST, 128), jnp.float32),
            pltpu.VMEM((_NB, _HIST, _EMBED), jnp.float32),
        ] + [pltpu.SemaphoreType.DMA] * (2 * _NB),
    )(_gather_body)
    return k(x, w_wide)


def kernel(x, W):
    w_wide = jnp.pad(W, ((0, 0), (0, 128 - _EMBED)))
    return _embed(x, w_wide)


# speedup vs baseline: 1.1659x; 1.1573x over previous
import functools

import jax
import jax.numpy as jnp
from jax import lax
from jax.experimental import pallas as pl
from jax.experimental.pallas import tpu as pltpu
from jax.experimental.pallas import tpu_sc as plsc

_BATCH = 16384
_HIST = 50
_EMBED = 32
_NC = 2
_NS = 16
_NW = _NC * _NS
_RPW = _BATCH // _NW
_XB = 128
_NBLK = _RPW // _XB
_NB = 8
_K = 6


def _gather_body(x_hbm, w_hbm, out_hbm, idx_v, rows_v, *sems):
    gsem = sems[:_NB]
    ssem = sems[_NB:]
    wid = lax.axis_index("s") * _NC + lax.axis_index("c")
    row0 = wid * _RPW

    for blk in range(_NBLK):
        base = row0 + blk * _XB

        def gather(g, b):
            pltpu.async_copy(w_hbm.at[idx_v.at[g]], rows_v.at[b], gsem[b])

        def wait_gather(b):
            pltpu.make_async_copy(w_hbm.at[idx_v.at[0]], rows_v.at[b],
                                  gsem[b]).wait()

        def store(g, b):
            pltpu.async_copy(rows_v.at[b], out_hbm.at[base + g], ssem[b])

        def wait_store(b):
            pltpu.make_async_copy(rows_v.at[b], out_hbm.at[0],
                                  ssem[b]).wait()

        def body(g, b, bk, with_ssem_wait):
            if with_ssem_wait:
                wait_store(bk)
            gather(g + _K, bk)
            wait_gather(b)
            store(g, b)

        pltpu.sync_copy(x_hbm.at[pl.ds(base, _XB)], idx_v)
        for g in range(_K):
            gather(g, g)
        for g in range(_NB - _K):
            body(g, g % _NB, (g + _K) % _NB, False)
        for g in range(_NB - _K, _NB):
            body(g, g % _NB, (g + _K) % _NB, True)

        def outer(i, carry):
            for b in range(_NB):
                g = i * _NB + b
                body(g, b, (b + _K) % _NB, True)
            return carry

        lax.fori_loop(1, (_XB - _K) // _NB, outer, 0)

        for g in range(_XB - _NB, _XB - _K):
            body(g, g % _NB, (g + _K) % _NB, True)
        for g in range(_XB - _K, _XB):
            wait_gather(g % _NB)
            store(g, g % _NB)
        for g in range(_XB - _NB, _XB):
            wait_store(g % _NB)


@jax.jit
def _embed(x, w_wide):
    k = functools.partial(
        pl.kernel,
        mesh=plsc.VectorSubcoreMesh(core_axis_name="c", subcore_axis_name="s"),
        out_type=jax.ShapeDtypeStruct((_BATCH, _HIST, 128), jnp.float32),
        scratch_types=[
            pltpu.VMEM((_XB, _HIST), jnp.int32),
            pltpu.VMEM((_NB, _HIST, 128), jnp.float32),
        ] + [pltpu.SemaphoreType.DMA] * (2 * _NB),
    )(_gather_body)
    return k(x, w_wide)


def kernel(x, W):
    w_wide = jnp.pad(W, ((0, 0), (0, 128 - _EMBED)))
    return _embed(x, w_wide)[:, :, :_EMBED]
